# Initial kernel scaffold; baseline (speedup 1.0000x reference)
import jax
import jax.numpy as jnp
from jax.experimental import pallas as pl


def kernel(pos, edge_index, cell, frac_coords):
    raise NotImplementedError("write your pallas kernel here")


# speedup vs baseline: 11.7514x; 11.7514x over previous
import dataclasses
import functools

import jax
import jax.numpy as jnp
from jax import lax
from jax.experimental import pallas as pl
from jax.experimental.pallas import tpu as pltpu
from jax.experimental.pallas import tpu_sc as plsc

_NC = 2
_NS = 16
_NW = _NC * _NS
_L = 16


def _pick_chunk(e_per_w: int) -> int:
    for ch in range(2048, 0, -16):
        if e_per_w % ch == 0:
            return ch
    raise ValueError(f"no chunk divides {e_per_w}")


@functools.partial(jax.jit, static_argnames=("e", "n"))
def _sc_periodic_distance(table, row, col, cellflat, *, e, n):
    del n
    assert e % _NW == 0
    e_per_w = e // _NW
    ch = _pick_chunk(e_per_w)
    n_chunks = e_per_w // ch

    mesh = plsc.VectorSubcoreMesh(core_axis_name="c", subcore_axis_name="s")

    cp = pltpu.CompilerParams()
    if "needs_layout_passes" in pltpu.CompilerParams.__dataclass_fields__:
        cp = dataclasses.replace(cp, needs_layout_passes=False)
    if "use_tc_tiling_on_sc" in pltpu.CompilerParams.__dataclass_fields__:
        cp = dataclasses.replace(cp, use_tc_tiling_on_sc=False)

    @functools.partial(
        pl.kernel,
        compiler_params=cp,
        out_type=(
            jax.ShapeDtypeStruct((e,), jnp.float32),
            jax.ShapeDtypeStruct((e, 3), jnp.float32),
        ),
        mesh=mesh,
        scratch_types=[
            pltpu.VMEM((ch,), jnp.int32),
            pltpu.VMEM((ch,), jnp.int32),
            pltpu.VMEM((ch, 8), jnp.float32),
            pltpu.VMEM((ch, 8), jnp.float32),
            pltpu.VMEM((ch,), jnp.float32),
            pltpu.VMEM((ch, 3), jnp.float32),
            pltpu.VMEM((16,), jnp.float32),
        ],
    )
    def k(table_hbm, row_hbm, col_hbm, cell_hbm, dist_hbm, delta_hbm,
          ridx_v, cidx_v, a_v, b_v, dist_v, delta_v, cell_v):
        wid = lax.axis_index("c") * _NS + lax.axis_index("s")
        base = wid * e_per_w
        pltpu.sync_copy(cell_hbm, cell_v)

        cv = cell_v[...]
        c00 = cv[0]
        c01 = cv[1]
        c02 = cv[2]
        c10 = cv[3]
        c11 = cv[4]
        c12 = cv[5]
        c20 = cv[6]
        c21 = cv[7]
        c22 = cv[8]

        lanes = lax.iota(jnp.int32, _L)
        half = jnp.float32(0.5)
        one = jnp.float32(1.0)

        def minimage(d):
            d = jnp.where(d > half, d - one, d)
            return jnp.where(d < -half, d + one, d)

        @pl.loop(0, n_chunks)
        def _(cidx):
            off = base + cidx * ch
            pltpu.sync_copy(row_hbm.at[pl.ds(off, ch)], ridx_v)
            pltpu.sync_copy(col_hbm.at[pl.ds(off, ch)], cidx_v)
            pltpu.sync_copy(table_hbm.at[ridx_v], a_v)
            pltpu.sync_copy(table_hbm.at[cidx_v], b_v)

            @pl.loop(0, ch, step=_L)
            def _(o):
                r = lanes + o
                k0 = jnp.zeros((_L,), jnp.int32)
                k1 = k0 + 1
                k2 = k0 + 2
                ax = plsc.load_gather(a_v, [r, k0])
                ay = plsc.load_gather(a_v, [r, k1])
                az = plsc.load_gather(a_v, [r, k2])
                bx = plsc.load_gather(b_v, [r, k0])
                by = plsc.load_gather(b_v, [r, k1])
                bz = plsc.load_gather(b_v, [r, k2])
                mx = minimage(ax - bx)
                my = minimage(ay - by)
                mz = minimage(az - bz)
                dx = mx * c00 + my * c10 + mz * c20
                dy = mx * c01 + my * c11 + mz * c21
                dz = mx * c02 + my * c12 + mz * c22
                t = dx * dx + dy * dy + dz * dz + jnp.float32(1e-8)
                i = plsc.bitcast(t, jnp.int32)
                i = jnp.int32(0x5F3759DF) - (i >> 1)
                y = plsc.bitcast(i, jnp.float32)
                yh = t * half
                y = y * (jnp.float32(1.5) - yh * y * y)
                y = y * (jnp.float32(1.5) - yh * y * y)
                y = y * (jnp.float32(1.5) - yh * y * y)
                dist_v[pl.ds(o, _L)] = t * y
                plsc.store_scatter(delta_v, [r, k0], dx)
                plsc.store_scatter(delta_v, [r, k1], dy)
                plsc.store_scatter(delta_v, [r, k2], dz)

            pltpu.sync_copy(dist_v, dist_hbm.at[pl.ds(off, ch)])
            pltpu.sync_copy(delta_v, delta_hbm.at[pl.ds(off, ch), :])

    return k(table, row, col, cellflat)


def kernel(pos, edge_index, cell, frac_coords):
    del pos
    n = frac_coords.shape[0]
    e = edge_index.shape[1]
    table = jnp.concatenate(
        [frac_coords.astype(jnp.float32),
         jnp.zeros((n, 5), jnp.float32)], axis=1)
    cellflat = jnp.concatenate(
        [cell.astype(jnp.float32).reshape(9), jnp.zeros((7,), jnp.float32)])
    row = edge_index[0]
    col = edge_index[1]
    dist, delta = _sc_periodic_distance(table, row, col, cellflat, e=e, n=n)
    return dist, delta


# speedup vs baseline: 14.3260x; 1.2191x over previous
import dataclasses
import functools

import jax
import jax.numpy as jnp
from jax import lax
from jax.experimental import pallas as pl
from jax.experimental.pallas import tpu as pltpu
from jax.experimental.pallas import tpu_sc as plsc

_NC = 2
_NS = 16
_NW = _NC * _NS
_L = 16


def _pick_chunk(e_per_w: int) -> int:
    for ch in range(2048, 0, -16):
        if e_per_w % ch == 0:
            return ch
    raise ValueError(f"no chunk divides {e_per_w}")


def _compiler_params():
    cp = pltpu.CompilerParams()
    fields = pltpu.CompilerParams.__dataclass_fields__
    if "needs_layout_passes" in fields:
        cp = dataclasses.replace(cp, needs_layout_passes=False)
    if "use_tc_tiling_on_sc" in fields:
        cp = dataclasses.replace(cp, use_tc_tiling_on_sc=False)
    return cp


@functools.partial(jax.jit, static_argnames=("e", "n"))
def _sc_periodic_distance(table, edge_index, cell, *, e, n):
    del n
    assert e % _NW == 0
    e_per_w = e // _NW
    ch = _pick_chunk(e_per_w)
    n_chunks = e_per_w // ch

    mesh = plsc.VectorSubcoreMesh(core_axis_name="c", subcore_axis_name="s")

    @functools.partial(
        pl.kernel,
        compiler_params=_compiler_params(),
        out_type=(
            jax.ShapeDtypeStruct((e,), jnp.float32),
            jax.ShapeDtypeStruct((e, 3), jnp.float32),
        ),
        mesh=mesh,
        scratch_types=[
            [pltpu.VMEM((ch,), jnp.int32) for _ in range(2)],
            [pltpu.VMEM((ch,), jnp.int32) for _ in range(2)],
            [pltpu.VMEM((ch, 8), jnp.float32) for _ in range(2)],
            [pltpu.VMEM((ch, 8), jnp.float32) for _ in range(2)],
            [pltpu.VMEM((ch,), jnp.float32) for _ in range(2)],
            [pltpu.VMEM((ch, 3), jnp.float32) for _ in range(2)],
            pltpu.VMEM((3, 3), jnp.float32),
            [pltpu.SemaphoreType.DMA for _ in range(2)],
            [pltpu.SemaphoreType.DMA for _ in range(2)],
            [pltpu.SemaphoreType.DMA for _ in range(2)],
        ],
    )
    def k(table_hbm, edge_hbm, cell_hbm, dist_hbm, delta_hbm,
          ridx_v, cidx_v, a_v, b_v, dist_v, delta_v, cell_v,
          isem, gsem, osem):
        wid = lax.axis_index("c") * _NS + lax.axis_index("s")
        base = wid * e_per_w
        pltpu.sync_copy(cell_hbm, cell_v)

        lanes = lax.iota(jnp.int32, _L)
        crow = jnp.where(lanes < 9, lanes, 0) // 3
        ccol = jnp.where(lanes < 9, lanes, 0) % 3
        cv = plsc.load_gather(cell_v, [crow, ccol])
        c00 = cv[0]
        c01 = cv[1]
        c02 = cv[2]
        c10 = cv[3]
        c11 = cv[4]
        c12 = cv[5]
        c20 = cv[6]
        c21 = cv[7]
        c22 = cv[8]

        half = jnp.float32(0.5)
        one = jnp.float32(1.0)

        def minimage(d):
            d = jnp.where(d > half, d - one, d)
            return jnp.where(d < -half, d + one, d)

        def issue_idx(i, b):
            off = base + i * ch
            pltpu.async_copy(edge_hbm.at[0, pl.ds(off, ch)], ridx_v[b], isem[b])
            pltpu.async_copy(edge_hbm.at[1, pl.ds(off, ch)], cidx_v[b], isem[b])

        def wait_idx(b):
            pltpu.make_async_copy(edge_hbm.at[0, pl.ds(0, ch)], ridx_v[b],
                                  isem[b]).wait()
            pltpu.make_async_copy(edge_hbm.at[1, pl.ds(0, ch)], cidx_v[b],
                                  isem[b]).wait()

        def issue_gather(b):
            pltpu.async_copy(table_hbm.at[ridx_v[b]], a_v[b], gsem[b])
            pltpu.async_copy(table_hbm.at[cidx_v[b]], b_v[b], gsem[b])

        def wait_gather(b):
            pltpu.make_async_copy(table_hbm.at[ridx_v[b]], a_v[b],
                                  gsem[b]).wait()
            pltpu.make_async_copy(table_hbm.at[cidx_v[b]], b_v[b],
                                  gsem[b]).wait()

        def issue_out(i, b):
            off = base + i * ch
            pltpu.async_copy(dist_v[b], dist_hbm.at[pl.ds(off, ch)], osem[b])
            pltpu.async_copy(delta_v[b], delta_hbm.at[pl.ds(off, ch), :],
                             osem[b])

        def wait_out(b):
            pltpu.make_async_copy(dist_v[b], dist_hbm.at[pl.ds(0, ch)],
                                  osem[b]).wait()
            pltpu.make_async_copy(delta_v[b], delta_hbm.at[pl.ds(0, ch), :],
                                  osem[b]).wait()

        def compute(b):
            @pl.loop(0, ch, step=_L)
            def _(o):
                r = lanes + o
                k0 = jnp.zeros((_L,), jnp.int32)
                k1 = k0 + 1
                k2 = k0 + 2
                ax = plsc.load_gather(a_v[b], [r, k0])
                ay = plsc.load_gather(a_v[b], [r, k1])
                az = plsc.load_gather(a_v[b], [r, k2])
                bx = plsc.load_gather(b_v[b], [r, k0])
                by = plsc.load_gather(b_v[b], [r, k1])
                bz = plsc.load_gather(b_v[b], [r, k2])
                mx = minimage(ax - bx)
                my = minimage(ay - by)
                mz = minimage(az - bz)
                dx = mx * c00 + my * c10 + mz * c20
                dy = mx * c01 + my * c11 + mz * c21
                dz = mx * c02 + my * c12 + mz * c22
                t = dx * dx + dy * dy + dz * dz + jnp.float32(1e-8)
                i = plsc.bitcast(t, jnp.int32)
                i = jnp.int32(0x5F3759DF) - (i >> 1)
                y = plsc.bitcast(i, jnp.float32)
                yh = t * half
                y = y * (jnp.float32(1.5) - yh * y * y)
                y = y * (jnp.float32(1.5) - yh * y * y)
                y = y * (jnp.float32(1.5) - yh * y * y)
                dist_v[b][pl.ds(o, _L)] = t * y
                plsc.store_scatter(delta_v[b], [r, k0], dx)
                plsc.store_scatter(delta_v[b], [r, k1], dy)
                plsc.store_scatter(delta_v[b], [r, k2], dz)

        issue_idx(0, 0)
        if n_chunks > 1:
            issue_idx(1, 1)
        wait_idx(0)
        issue_gather(0)

        assert n_chunks % 2 == 0

        @pl.loop(0, n_chunks, step=2)
        def _(i0):
            for p in range(2):
                b = p
                nb = 1 - p
                i = i0 + p
                wait_gather(b)

                @pl.when(i + 1 < n_chunks)
                def _():
                    wait_idx(nb)
                    issue_gather(nb)

                @pl.when(i + 2 < n_chunks)
                def _():
                    issue_idx(i + 2, b)

                @pl.when(i >= 2)
                def _():
                    wait_out(b)

                compute(b)
                issue_out(i, b)

        wait_out(0)
        if n_chunks > 1:
            wait_out(1)

    return k(table, edge_index, cell)


def kernel(pos, edge_index, cell, frac_coords):
    del pos
    n = frac_coords.shape[0]
    e = edge_index.shape[1]
    table = jnp.concatenate(
        [frac_coords.astype(jnp.float32),
         jnp.zeros((n, 5), jnp.float32)], axis=1)
    dist, delta = _sc_periodic_distance(
        table, edge_index, cell.astype(jnp.float32), e=e, n=n)
    return dist, delta
